# baseline (device time: 236187 ns/iter reference)
import jax
import jax.numpy as jnp
from jax import lax
from jax.experimental import pallas as pl
from jax.experimental.pallas import tpu as pltpu

N_DEV = 32
M = 4096
M_PER = M // N_DEV
N_HOPS = N_DEV - 1
N_LANES = 8
LANE_N = 2048 // N_LANES


def _ring_tables():
    import distributed_mesh_v7x as dm

    mesh = dm.get_mesh("i", world_size=N_DEV)
    mesh_coords = [tuple(d.coords) for d in mesh.devices]
    coord_to_logical = {c: i for i, c in enumerate(mesh_coords)}

    path_yz = []
    for y in range(4):
        zs = range(4) if y % 2 == 0 else range(3, -1, -1)
        path_yz.extend((y, z) for z in zs)
    ring_coords = [(0, y, z) for (y, z) in path_yz]
    ring_coords += [(1, y, z) for (y, z) in reversed(path_yz)]

    for a, b in zip(ring_coords, ring_coords[1:] + ring_coords[:1]):
        assert sum(abs(u - v) for u, v in zip(a, b)) == 1, (a, b)
    L = [coord_to_logical[c] for c in ring_coords]
    P = [0] * N_DEV
    for pos, logical in enumerate(L):
        P[logical] = pos
    return L, P


def kernel(x, w_mat, scale_x, scale_w):
    m, k_per = x.shape
    _, n = w_mat.shape
    assert m == M and n == 2048

    L, P = _ring_tables()
    L_arr = jnp.asarray(L, dtype=jnp.int32)
    P_arr = jnp.asarray(P, dtype=jnp.int32)

    my_l = lax.axis_index("i").astype(jnp.int32)
    r = P_arr[my_l]
    left = L_arr[(r - 1) % N_DEV]
    right = L_arr[(r + 1) % N_DEV]
    hops = jnp.arange(N_HOPS, dtype=jnp.int32)
    cw_blocks = L_arr[(r - 2 - hops) % N_DEV]
    ccw_blocks = L_arr[(r + 2 + hops) % N_DEV]
    params = jnp.concatenate(
        [jnp.stack([left, right]), cw_blocks, ccw_blocks]
    ).astype(jnp.int32)

    half = N_LANES // 2
    LANES = [
        (li * LANE_N, 1, 0, 2) if li < half else
        (li * LANE_N, 0, 1, 2 + N_HOPS)
        for li in range(N_LANES)
    ]
    LANE_ORDER = tuple(
        li for pair in zip(range(half), range(half, N_LANES)) for li in pair
    )

    def body(params_ref, x_ref, w_ref, sx_ref, sw_ref, out_ref,
             comm_ref, send_sems, recv_sems, credit_sems):
        nbr = [params_ref[0], params_ref[1]]

        barrier = pltpu.get_barrier_semaphore()
        for d in nbr:
            pl.semaphore_signal(
                barrier, inc=1, device_id=(d,),
                device_id_type=pl.DeviceIdType.MESH,
            )
        pl.semaphore_wait(barrier, 2)

        def partial_block(b, col0):
            xs = x_ref[pl.ds(b * M_PER, M_PER), :]
            return lax.dot_general(
                xs, w_ref[:, col0:col0 + LANE_N], (((1,), (0,)), ((), ())),
                preferred_element_type=jnp.int32,
            )

        def mk(li, src_slot, dst_dev):
            dst_slot = 1 - src_slot
            return pltpu.make_async_remote_copy(
                src_ref=comm_ref.at[li, src_slot],
                dst_ref=comm_ref.at[li, dst_slot],
                send_sem=send_sems.at[li, src_slot],
                recv_sem=recv_sems.at[li, dst_slot],
                device_id=(dst_dev,),
                device_id_type=pl.DeviceIdType.MESH,
            )

        descs = [[None] * N_HOPS for _ in range(N_LANES)]
        for li in LANE_ORDER:
            col0, dsti, srci, base = LANES[li]
            comm_ref[li, 0] = partial_block(params_ref[srci], col0)
            d = mk(li, 0, nbr[dsti])
            descs[li][0] = d
            d.start()

        for h in range(N_HOPS):
            rcv = (h + 1) % 2
            for li in LANE_ORDER:
                col0, dsti, srci, base = LANES[li]
                d = descs[li][h]
                pb = partial_block(params_ref[base + h], col0)
                d.wait_recv()
                if h < N_HOPS - 1:
                    comm_ref[li, rcv] = comm_ref[li, rcv] + pb
                    d.wait_send()
                    pl.semaphore_signal(
                        credit_sems.at[li], inc=1, device_id=(nbr[srci],),
                        device_id_type=pl.DeviceIdType.MESH,
                    )
                    pl.semaphore_wait(credit_sems.at[li], 1)
                    nd = mk(li, rcv, nbr[dsti])
                    descs[li][h + 1] = nd
                    nd.start()
                else:
                    acc = comm_ref[li, rcv] + pb
                    y = acc.astype(jnp.float32) * (sx_ref[0] * sw_ref[0])
                    yc = jnp.clip(y, -60.0, 60.0)
                    out_ref[:, col0:col0 + LANE_N] = y / (1.0 + jnp.exp(-yc))
                    d.wait_send()

    return pl.pallas_call(
        body,
        out_shape=jax.ShapeDtypeStruct((M_PER, n), jnp.float32),
        in_specs=[
            pl.BlockSpec(memory_space=pltpu.SMEM),
            pl.BlockSpec(memory_space=pltpu.VMEM),
            pl.BlockSpec(memory_space=pltpu.VMEM),
            pl.BlockSpec(memory_space=pltpu.SMEM),
            pl.BlockSpec(memory_space=pltpu.SMEM),
        ],
        out_specs=pl.BlockSpec(memory_space=pltpu.VMEM),
        scratch_shapes=[
            pltpu.VMEM((N_LANES, 2, M_PER, LANE_N), jnp.int32),
            pltpu.SemaphoreType.DMA((N_LANES, 2)),
            pltpu.SemaphoreType.DMA((N_LANES, 2)),
            pltpu.SemaphoreType.REGULAR((N_LANES,)),
        ],
        compiler_params=pltpu.CompilerParams(collective_id=0),
    )(params, x, w_mat, scale_x, scale_w)


# device time: 185474 ns/iter; 1.2734x vs baseline; 1.2734x over previous
import jax
import jax.numpy as jnp
from jax import lax
from jax.experimental import pallas as pl
from jax.experimental.pallas import tpu as pltpu

N_DEV = 32
M = 4096
M_PER = M // N_DEV
N_HOPS = N_DEV - 1
N_LANES = 4
LANE_N = 2048 // N_LANES


def _ring_tables():
    import distributed_mesh_v7x as dm

    mesh = dm.get_mesh("i", world_size=N_DEV)
    mesh_coords = [tuple(d.coords) for d in mesh.devices]
    coord_to_logical = {c: i for i, c in enumerate(mesh_coords)}

    path_yz = []
    for y in range(4):
        zs = range(4) if y % 2 == 0 else range(3, -1, -1)
        path_yz.extend((y, z) for z in zs)
    ring_coords = [(0, y, z) for (y, z) in path_yz]
    ring_coords += [(1, y, z) for (y, z) in reversed(path_yz)]

    for a, b in zip(ring_coords, ring_coords[1:] + ring_coords[:1]):
        assert sum(abs(u - v) for u, v in zip(a, b)) == 1, (a, b)
    L = [coord_to_logical[c] for c in ring_coords]
    P = [0] * N_DEV
    for pos, logical in enumerate(L):
        P[logical] = pos
    return L, P


def kernel(x, w_mat, scale_x, scale_w):
    m, k_per = x.shape
    _, n = w_mat.shape
    assert m == M and n == 2048

    L, P = _ring_tables()
    L_arr = jnp.asarray(L, dtype=jnp.int32)
    P_arr = jnp.asarray(P, dtype=jnp.int32)

    my_l = lax.axis_index("i").astype(jnp.int32)
    r = P_arr[my_l]
    left = L_arr[(r - 1) % N_DEV]
    right = L_arr[(r + 1) % N_DEV]
    hops = jnp.arange(N_HOPS, dtype=jnp.int32)
    cw_blocks = L_arr[(r - 2 - hops) % N_DEV]
    ccw_blocks = L_arr[(r + 2 + hops) % N_DEV]
    params = jnp.concatenate(
        [jnp.stack([left, right]), cw_blocks, ccw_blocks]
    ).astype(jnp.int32)

    LANES = [
        (0 * LANE_N, 1, 0, 2),
        (1 * LANE_N, 1, 0, 2),
        (2 * LANE_N, 0, 1, 2 + N_HOPS),
        (3 * LANE_N, 0, 1, 2 + N_HOPS),
    ]
    LANE_ORDER = (0, 2, 1, 3)

    def body(params_ref, x_ref, w_ref, sx_ref, sw_ref, out_ref,
             comm_ref, send_sems, recv_sems, credit_sems):
        nbr = [params_ref[0], params_ref[1]]

        barrier = pltpu.get_barrier_semaphore()
        for d in nbr:
            pl.semaphore_signal(
                barrier, inc=1, device_id=(d,),
                device_id_type=pl.DeviceIdType.MESH,
            )
        pl.semaphore_wait(barrier, 2)

        def partial_block(b, col0):
            xs = x_ref[pl.ds(b * M_PER, M_PER), :]
            return lax.dot_general(
                xs, w_ref[:, col0:col0 + LANE_N], (((1,), (0,)), ((), ())),
                preferred_element_type=jnp.int32,
            )

        def mk(li, src_slot, dst_dev):
            dst_slot = 1 - src_slot
            return pltpu.make_async_remote_copy(
                src_ref=comm_ref.at[li, src_slot],
                dst_ref=comm_ref.at[li, dst_slot],
                send_sem=send_sems.at[li, src_slot],
                recv_sem=recv_sems.at[li, dst_slot],
                device_id=(dst_dev,),
                device_id_type=pl.DeviceIdType.MESH,
            )

        descs = [[None] * N_HOPS for _ in range(N_LANES)]
        for li in LANE_ORDER:
            col0, dsti, srci, base = LANES[li]
            comm_ref[li, 0] = partial_block(params_ref[srci], col0)
            d = mk(li, 0, nbr[dsti])
            descs[li][0] = d
            d.start()

        for h in range(N_HOPS):
            rcv = (h + 1) % 2
            for li in LANE_ORDER:
                col0, dsti, srci, base = LANES[li]
                d = descs[li][h]
                pb = partial_block(params_ref[base + h], col0)
                d.wait_recv()
                if h < N_HOPS - 1:
                    comm_ref[li, rcv] = comm_ref[li, rcv] + pb
                    d.wait_send()
                    pl.semaphore_signal(
                        credit_sems.at[li], inc=1, device_id=(nbr[srci],),
                        device_id_type=pl.DeviceIdType.MESH,
                    )
                    pl.semaphore_wait(credit_sems.at[li], 1)
                    nd = mk(li, rcv, nbr[dsti])
                    descs[li][h + 1] = nd
                    nd.start()
                else:
                    acc = comm_ref[li, rcv] + pb
                    y = acc.astype(jnp.float32) * (sx_ref[0] * sw_ref[0])
                    yc = jnp.clip(y, -60.0, 60.0)
                    out_ref[:, col0:col0 + LANE_N] = y / (1.0 + jnp.exp(-yc))
                    d.wait_send()

    return pl.pallas_call(
        body,
        out_shape=jax.ShapeDtypeStruct((M_PER, n), jnp.float32),
        in_specs=[
            pl.BlockSpec(memory_space=pltpu.SMEM),
            pl.BlockSpec(memory_space=pltpu.VMEM),
            pl.BlockSpec(memory_space=pltpu.VMEM),
            pl.BlockSpec(memory_space=pltpu.SMEM),
            pl.BlockSpec(memory_space=pltpu.SMEM),
        ],
        out_specs=pl.BlockSpec(memory_space=pltpu.VMEM),
        scratch_shapes=[
            pltpu.VMEM((N_LANES, 2, M_PER, LANE_N), jnp.int32),
            pltpu.SemaphoreType.DMA((N_LANES, 2)),
            pltpu.SemaphoreType.DMA((N_LANES, 2)),
            pltpu.SemaphoreType.REGULAR((N_LANES,)),
        ],
        compiler_params=pltpu.CompilerParams(collective_id=0),
    )(params, x, w_mat, scale_x, scale_w)
